# baseline (speedup 1.0000x reference)
import functools

import jax
import jax.numpy as jnp
from jax import lax
from jax.experimental import pallas as pl
from jax.experimental.pallas import tpu as pltpu
from jax.experimental.pallas import tpu_sc as plsc

_N_LEVELS = 16
_NFEAT = 2
_LOG2 = 19
_HASHMAP = 2 ** _LOG2
_MASK = _HASHMAP - 1
_BASE, _FINEST = 16, 512
_P2 = 2654435761 & _MASK
_P3 = 805459861 & _MASK
_N_POINTS = 262144
_NOUT = _N_LEVELS * _NFEAT

_NC, _NS, _L = 2, 16, 16
_NW = _NC * _NS
_NPT = _N_POINTS // _NW
_C = 1024
_NCHUNK = _NPT // _C
_G16 = _C // _L
_G8 = _C // 8


def _resolutions():
    growth = (_FINEST / _BASE) ** (1.0 / (_N_LEVELS - 1))
    return [int(_BASE * growth ** i) for i in range(_N_LEVELS)]


_RCHUNK = 8192


def _delinearize(flat):
    u = _RCHUNK // 128

    def body(i_ref, o_ref):
        for j in range(u):
            o_ref[pl.ds(j * 128, 128), :] = i_ref[:, j, :].T

    flat3 = flat.reshape(_NOUT, _N_POINTS // 128, 128)
    return pl.pallas_call(
        body,
        grid=(_N_POINTS // _RCHUNK,),
        in_specs=[pl.BlockSpec((_NOUT, u, 128), lambda i: (0, i, 0))],
        out_specs=pl.BlockSpec((_RCHUNK, _NOUT), lambda i: (i, 0)),
        out_shape=jax.ShapeDtypeStruct((_N_POINTS, _NOUT), jnp.float32),
    )(flat3)


def kernel(x, tables):
    tables_flat = tables.reshape(-1)
    x_flat = x.reshape(-1)
    res = _resolutions()
    mesh = plsc.VectorSubcoreMesh(core_axis_name="c", subcore_axis_name="s")

    @functools.partial(
        pl.kernel,
        out_type=jax.ShapeDtypeStruct((_N_POINTS * _NOUT,), jnp.float32),
        mesh=mesh,
        compiler_params=pltpu.CompilerParams(needs_layout_passes=False),
        scratch_types=[
            pltpu.VMEM((3 * _NPT,), jnp.float32),
            pltpu.VMEM((16 * _C,), jnp.int32),
            pltpu.VMEM((16 * _C,), jnp.float32),
            pltpu.VMEM((3 * _C,), jnp.float32),
            pltpu.VMEM((_C * _NOUT,), jnp.float32),
            pltpu.SemaphoreType.DMA,
        ],
    )
    def _k(x_hbm, tab_hbm, out_hbm, xbuf, idxbuf, featbuf, fracbuf, outbuf, sem):
        wid = lax.axis_index("s") * _NC + lax.axis_index("c")
        base_pt = wid * _NPT
        pltpu.sync_copy(x_hbm.at[pl.ds(base_pt * 3, _NPT * 3)], xbuf)
        lanes = lax.iota(jnp.int32, _L)
        lanes2 = lanes * 2
        dup = lax.shift_right_logical(lanes, 1)
        par = lax.bitwise_and(lanes, 1)
        par_c = par * _C

        def chunk_body(chunk, carry0):
            c0 = chunk * _C
            for lvl in range(_N_LEVELS):
                rf = float(res[lvl])
                lvl_off = lvl << _LOG2

                def pass_a(g, carry):
                    rows3 = (c0 + g * _L) * 3 + lanes * 3
                    xv = plsc.load_gather(xbuf, [rows3])
                    yv = plsc.load_gather(xbuf, [rows3 + 1])
                    zv = plsc.load_gather(xbuf, [rows3 + 2])
                    sx = xv * rf
                    sy = yv * rf
                    sz = zv * rf
                    fxi = sx.astype(jnp.int32)
                    fyi = sy.astype(jnp.int32)
                    fzi = sz.astype(jnp.int32)
                    q = g * _L
                    fracbuf[pl.ds(q, _L)] = sx - fxi.astype(jnp.float32)
                    fracbuf[pl.ds(_C + q, _L)] = sy - fyi.astype(jnp.float32)
                    fracbuf[pl.ds(2 * _C + q, _L)] = sz - fzi.astype(jnp.float32)
                    hx0 = fxi
                    hx1 = fxi + 1
                    hy0 = fyi * _P2
                    hy1 = (fyi + 1) * _P2
                    hz0 = fzi * _P3
                    hz1 = (fzi + 1) * _P3
                    corners = ((hx0, hy0, hz0), (hx1, hy0, hz0),
                               (hx0, hy1, hz0), (hx1, hy1, hz0),
                               (hx0, hy0, hz1), (hx1, hy0, hz1),
                               (hx0, hy1, hz1), (hx1, hy1, hz1))
                    for c, (hx, hy, hz) in enumerate(corners):
                        t = (((hx ^ hy ^ hz) & _MASK) | lvl_off) * 2
                        pos0 = 2 * (c * _C + q) + lanes2
                        plsc.store_scatter(idxbuf, [pos0], t)
                        plsc.store_scatter(idxbuf, [pos0 + 1], t + 1)
                    return carry

                lax.fori_loop(0, _G16, pass_a, 0)

                pltpu.async_copy(tab_hbm.at[idxbuf], featbuf, sem).wait()

                def pass_b(g, carry):
                    prow = g * 8 + dup
                    wx = plsc.load_gather(fracbuf, [prow])
                    wy = plsc.load_gather(fracbuf, [prow + _C])
                    wz = plsc.load_gather(fracbuf, [prow + 2 * _C])
                    ux = 1.0 - wx
                    uy = 1.0 - wy
                    uz = 1.0 - wz
                    pa = ux * uy
                    pb = wx * uy
                    pc = ux * wy
                    pd = wx * wy
                    ws = (pa * uz, pb * uz, pc * uz, pd * uz,
                          pa * wz, pb * wz, pc * wz, pd * wz)
                    q = g * _L
                    acc = ws[0] * featbuf[pl.ds(q, _L)]
                    for c in range(1, 8):
                        fv = featbuf[pl.ds(2 * c * _C + q, _L)]
                        acc = acc + ws[c] * fv
                    plsc.store_scatter(outbuf, [(2 * lvl * _C) + par_c + prow], acc)
                    return carry

                lax.fori_loop(0, _G8, pass_b, 0)

            for q in range(_NOUT):
                pltpu.sync_copy(
                    outbuf.at[pl.ds(q * _C, _C)],
                    out_hbm.at[pl.ds(q * _N_POINTS + base_pt + c0, _C)],
                )
            return carry0

        lax.fori_loop(0, _NCHUNK, chunk_body, 0)

    return _delinearize(_k(x_flat, tables_flat))


# speedup vs baseline: 4.1323x; 4.1323x over previous
import functools

import jax
import jax.numpy as jnp
from jax import lax
from jax.experimental import pallas as pl
from jax.experimental.pallas import tpu as pltpu
from jax.experimental.pallas import tpu_sc as plsc

_N_LEVELS = 16
_NFEAT = 2
_LOG2 = 19
_HASHMAP = 2 ** _LOG2
_MASK = _HASHMAP - 1
_BASE, _FINEST = 16, 512
_P2 = 2654435761 & _MASK
_P3 = 805459861 & _MASK
_BLKMASK = _MASK & ~127
_N_POINTS = 262144
_NOUT = _N_LEVELS * _NFEAT

_NC, _NS, _L = 2, 16, 16
_NW = _NC * _NS
_NPT = _N_POINTS // _NW
_C = 1024
_NCHUNK = _NPT // _C
_G16 = _C // _L
_G8 = _C // 8


def _resolutions():
    growth = (_FINEST / _BASE) ** (1.0 / (_N_LEVELS - 1))
    return [int(_BASE * growth ** i) for i in range(_N_LEVELS)]


_RCHUNK = 8192


def _delinearize(flat):
    u = _RCHUNK // 128

    def body(i_ref, o_ref):
        for j in range(u):
            o_ref[:, pl.ds(j * 128, 128)] = i_ref[:, j, :]

    flat3 = flat.reshape(_NOUT, _N_POINTS // 128, 128)
    out = pl.pallas_call(
        body,
        grid=(_N_POINTS // _RCHUNK,),
        in_specs=[pl.BlockSpec((_NOUT, u, 128), lambda i: (0, i, 0))],
        out_specs=pl.BlockSpec((_NOUT, _RCHUNK), lambda i: (0, i)),
        out_shape=jax.ShapeDtypeStruct((_NOUT, _N_POINTS), jnp.float32),
    )(flat3)
    return out.T


def kernel(x, tables):
    tables_flat = (
        tables.transpose(0, 2, 1)
        .reshape(_N_LEVELS, _NFEAT, _HASHMAP // 128, 128)
        .transpose(0, 2, 1, 3)
        .reshape(-1)
    )
    x_flat = x.reshape(-1)
    res = _resolutions()
    mesh = plsc.VectorSubcoreMesh(core_axis_name="c", subcore_axis_name="s")

    @functools.partial(
        pl.kernel,
        out_type=jax.ShapeDtypeStruct((_N_POINTS * _NOUT,), jnp.float32),
        mesh=mesh,
        compiler_params=pltpu.CompilerParams(needs_layout_passes=False),
        scratch_types=[
            pltpu.VMEM((3 * _NPT,), jnp.float32),
            pltpu.VMEM((16 * _C,), jnp.int32),
            pltpu.VMEM((16 * _C,), jnp.float32),
            pltpu.VMEM((3 * _C,), jnp.float32),
            pltpu.VMEM((_C * _NOUT,), jnp.float32),
            pltpu.SemaphoreType.DMA,
        ],
    )
    def _k(x_hbm, tab_hbm, out_hbm, xbuf, idxbuf, featbuf, fracbuf, outbuf, sem):
        wid = lax.axis_index("s") * _NC + lax.axis_index("c")
        base_pt = wid * _NPT
        pltpu.sync_copy(x_hbm.at[pl.ds(base_pt * 3, _NPT * 3)], xbuf)
        lanes = lax.iota(jnp.int32, _L)
        lanes2 = lanes * 2
        dup = lax.shift_right_logical(lanes, 1)
        par = lax.bitwise_and(lanes, 1)
        par_c = par * _C

        def chunk_body(chunk, carry0):
            c0 = chunk * _C
            for lvl in range(_N_LEVELS):
                rf = float(res[lvl])
                lvl_base = lvl << (_LOG2 + 1)

                def pass_a(g, carry):
                    rows3 = (c0 + g * _L) * 3 + lanes * 3
                    xv = plsc.load_gather(xbuf, [rows3])
                    yv = plsc.load_gather(xbuf, [rows3 + 1])
                    zv = plsc.load_gather(xbuf, [rows3 + 2])
                    sx = xv * rf
                    sy = yv * rf
                    sz = zv * rf
                    fxi = sx.astype(jnp.int32)
                    fyi = sy.astype(jnp.int32)
                    fzi = sz.astype(jnp.int32)
                    q = g * _L
                    fracbuf[pl.ds(q, _L)] = sx - fxi.astype(jnp.float32)
                    fracbuf[pl.ds(_C + q, _L)] = sy - fyi.astype(jnp.float32)
                    fracbuf[pl.ds(2 * _C + q, _L)] = sz - fzi.astype(jnp.float32)
                    hx0 = fxi
                    hx1 = fxi + 1
                    hy0 = fyi * _P2
                    hy1 = (fyi + 1) * _P2
                    hz0 = fzi * _P3
                    hz1 = (fzi + 1) * _P3
                    corners = ((hx0, hy0, hz0), (hx1, hy0, hz0),
                               (hx0, hy1, hz0), (hx1, hy1, hz0),
                               (hx0, hy0, hz1), (hx1, hy0, hz1),
                               (hx0, hy1, hz1), (hx1, hy1, hz1))
                    for c, (hx, hy, hz) in enumerate(corners):
                        h = (hx ^ hy ^ hz) & _MASK
                        e0 = lvl_base + h + (h & _BLKMASK)
                        pos0 = 2 * (c * _C + q) + lanes2
                        plsc.store_scatter(idxbuf, [pos0], e0)
                        plsc.store_scatter(idxbuf, [pos0 + 1], e0 + 128)
                    return carry

                lax.fori_loop(0, _G16, pass_a, 0)

                pltpu.async_copy(tab_hbm.at[idxbuf], featbuf, sem).wait()

                def pass_b(g, carry):
                    prow = g * 8 + dup
                    wx = plsc.load_gather(fracbuf, [prow])
                    wy = plsc.load_gather(fracbuf, [prow + _C])
                    wz = plsc.load_gather(fracbuf, [prow + 2 * _C])
                    ux = 1.0 - wx
                    uy = 1.0 - wy
                    uz = 1.0 - wz
                    pa = ux * uy
                    pb = wx * uy
                    pc = ux * wy
                    pd = wx * wy
                    ws = (pa * uz, pb * uz, pc * uz, pd * uz,
                          pa * wz, pb * wz, pc * wz, pd * wz)
                    q = g * _L
                    acc = ws[0] * featbuf[pl.ds(q, _L)]
                    for c in range(1, 8):
                        fv = featbuf[pl.ds(2 * c * _C + q, _L)]
                        acc = acc + ws[c] * fv
                    plsc.store_scatter(outbuf, [(2 * lvl * _C) + par_c + prow], acc)
                    return carry

                lax.fori_loop(0, _G8, pass_b, 0)

            for q in range(_NOUT):
                pltpu.sync_copy(
                    outbuf.at[pl.ds(q * _C, _C)],
                    out_hbm.at[pl.ds(q * _N_POINTS + base_pt + c0, _C)],
                )
            return carry0

        lax.fori_loop(0, _NCHUNK, chunk_body, 0)

    return _delinearize(_k(x_flat, tables_flat))


# speedup vs baseline: 4.9893x; 1.2074x over previous
import functools

import jax
import jax.numpy as jnp
from jax import lax
from jax.experimental import pallas as pl
from jax.experimental.pallas import tpu as pltpu
from jax.experimental.pallas import tpu_sc as plsc

_N_LEVELS = 16
_NFEAT = 2
_LOG2 = 19
_HASHMAP = 2 ** _LOG2
_MASK = _HASHMAP - 1
_BASE, _FINEST = 16, 512
_P2 = 2654435761 & _MASK
_P3 = 805459861 & _MASK
_BLKMASK = _MASK & ~127
_N_POINTS = 262144
_NOUT = _N_LEVELS * _NFEAT

_NC, _NS, _L = 2, 16, 16
_NW = _NC * _NS
_NPT = _N_POINTS // _NW
_C = 1024
_NCHUNK = _NPT // _C
_G16 = _C // _L
_G8 = _C // 8


def _resolutions():
    growth = (_FINEST / _BASE) ** (1.0 / (_N_LEVELS - 1))
    return [int(_BASE * growth ** i) for i in range(_N_LEVELS)]


_RCHUNK = 8192


def _delinearize(flat):
    u = _RCHUNK // 128

    def body(i_ref, o_ref):
        for j in range(u):
            o_ref[:, pl.ds(j * 128, 128)] = i_ref[:, j, :]

    flat3 = flat.reshape(_NOUT, _N_POINTS // 128, 128)
    out = pl.pallas_call(
        body,
        grid=(_N_POINTS // _RCHUNK,),
        in_specs=[pl.BlockSpec((_NOUT, u, 128), lambda i: (0, i, 0))],
        out_specs=pl.BlockSpec((_NOUT, _RCHUNK), lambda i: (0, i)),
        out_shape=jax.ShapeDtypeStruct((_NOUT, _N_POINTS), jnp.float32),
    )(flat3)
    return out.T


def kernel(x, tables):
    tables_flat = (
        tables.transpose(0, 2, 1)
        .reshape(_N_LEVELS, _NFEAT, _HASHMAP // 128, 128)
        .transpose(0, 2, 1, 3)
        .reshape(-1)
    )
    x_flat = x.reshape(-1)
    res = _resolutions()
    mesh = plsc.VectorSubcoreMesh(core_axis_name="c", subcore_axis_name="s")

    @functools.partial(
        pl.kernel,
        out_type=jax.ShapeDtypeStruct((_N_POINTS * _NOUT,), jnp.float32),
        mesh=mesh,
        compiler_params=pltpu.CompilerParams(needs_layout_passes=False),
        scratch_types=[
            pltpu.VMEM((3 * _C,), jnp.float32),
            pltpu.VMEM((16 * _C,), jnp.int32),
            pltpu.VMEM((16 * _C,), jnp.int32),
            pltpu.VMEM((16 * _C,), jnp.float32),
            pltpu.VMEM((16 * _C,), jnp.float32),
            pltpu.VMEM((3 * _C,), jnp.float32),
            pltpu.VMEM((3 * _C,), jnp.float32),
            pltpu.VMEM((_C * _NOUT,), jnp.float32),
            pltpu.SemaphoreType.DMA,
            pltpu.SemaphoreType.DMA,
            pltpu.SemaphoreType.DMA,
        ],
    )
    def _k(x_hbm, tab_hbm, out_hbm, xbuf, idxb0, idxb1, featb0, featb1,
           fracb0, fracb1, outbuf, gsem0, gsem1, osem):
        wid = lax.axis_index("s") * _NC + lax.axis_index("c")
        base_pt = wid * _NPT
        idxbufs = (idxb0, idxb1)
        featbufs = (featb0, featb1)
        fracbufs = (fracb0, fracb1)
        gsems = (gsem0, gsem1)
        lanes = lax.iota(jnp.int32, _L)
        lanes2 = lanes * 2
        lanes3 = lanes * 3
        dup = lax.shift_right_logical(lanes, 1)
        par = lax.bitwise_and(lanes, 1)
        par_c = par * _C

        def make_pass_a(lvl, idxb, fracb):
            rf = float(res[lvl])
            lvl_base = lvl << (_LOG2 + 1)

            def pass_a(g, carry):
                rows3 = (g * _L) * 3 + lanes3
                xv = plsc.load_gather(xbuf, [rows3])
                yv = plsc.load_gather(xbuf, [rows3 + 1])
                zv = plsc.load_gather(xbuf, [rows3 + 2])
                sx = xv * rf
                sy = yv * rf
                sz = zv * rf
                fxi = sx.astype(jnp.int32)
                fyi = sy.astype(jnp.int32)
                fzi = sz.astype(jnp.int32)
                q = g * _L
                fracb[pl.ds(q, _L)] = sx - fxi.astype(jnp.float32)
                fracb[pl.ds(_C + q, _L)] = sy - fyi.astype(jnp.float32)
                fracb[pl.ds(2 * _C + q, _L)] = sz - fzi.astype(jnp.float32)
                hx0 = fxi
                hx1 = fxi + 1
                hy0 = fyi * _P2
                hy1 = (fyi + 1) * _P2
                hz0 = fzi * _P3
                hz1 = (fzi + 1) * _P3
                corners = ((hx0, hy0, hz0), (hx1, hy0, hz0),
                           (hx0, hy1, hz0), (hx1, hy1, hz0),
                           (hx0, hy0, hz1), (hx1, hy0, hz1),
                           (hx0, hy1, hz1), (hx1, hy1, hz1))
                for c, (hx, hy, hz) in enumerate(corners):
                    h = (hx ^ hy ^ hz) & _MASK
                    e0 = lvl_base + h + (h & _BLKMASK)
                    pos0 = 2 * (c * _C + q) + lanes2
                    plsc.store_scatter(idxb, [pos0], e0)
                    plsc.store_scatter(idxb, [pos0 + 1], e0 + 128)
                return carry

            return pass_a

        def make_pass_b(lvl, featb, fracb):
            def pass_b(g, carry):
                prow = g * 8 + dup
                wx = plsc.load_gather(fracb, [prow])
                wy = plsc.load_gather(fracb, [prow + _C])
                wz = plsc.load_gather(fracb, [prow + 2 * _C])
                ux = 1.0 - wx
                uy = 1.0 - wy
                uz = 1.0 - wz
                pa = ux * uy
                pb = wx * uy
                pc = ux * wy
                pd = wx * wy
                ws = (pa * uz, pb * uz, pc * uz, pd * uz,
                      pa * wz, pb * wz, pc * wz, pd * wz)
                q = g * _L
                acc = ws[0] * featb[pl.ds(q, _L)]
                for c in range(1, 8):
                    fv = featb[pl.ds(2 * c * _C + q, _L)]
                    acc = acc + ws[c] * fv
                plsc.store_scatter(outbuf, [(2 * lvl * _C) + par_c + prow], acc)
                return carry

            return pass_b

        def chunk_body(chunk, carry0):
            c0 = chunk * _C
            pltpu.sync_copy(x_hbm.at[pl.ds((base_pt + c0) * 3, _C * 3)], xbuf)

            ghandles = [None, None]
            ohandles = []

            def emit_out(lvl):
                for t in (0, 1):
                    q = 2 * lvl + t
                    ohandles.append(pltpu.async_copy(
                        outbuf.at[pl.ds(q * _C, _C)],
                        out_hbm.at[pl.ds(q * _N_POINTS + base_pt + c0, _C)],
                        osem,
                    ))

            lax.fori_loop(0, _G16, make_pass_a(0, idxbufs[0], fracbufs[0]), 0)
            ghandles[0] = pltpu.async_copy(
                tab_hbm.at[idxbufs[0]], featbufs[0], gsems[0])
            for lvl in range(1, _N_LEVELS):
                b = lvl & 1
                pb_ = b ^ 1
                lax.fori_loop(0, _G16, make_pass_a(lvl, idxbufs[b], fracbufs[b]), 0)
                ghandles[b] = pltpu.async_copy(
                    tab_hbm.at[idxbufs[b]], featbufs[b], gsems[b])
                ghandles[pb_].wait()
                lax.fori_loop(0, _G8, make_pass_b(lvl - 1, featbufs[pb_], fracbufs[pb_]), 0)
                emit_out(lvl - 1)
            last = (_N_LEVELS - 1) & 1
            ghandles[last].wait()
            lax.fori_loop(0, _G8, make_pass_b(_N_LEVELS - 1, featbufs[last], fracbufs[last]), 0)
            emit_out(_N_LEVELS - 1)
            for h in ohandles:
                h.wait()
            return carry0

        lax.fori_loop(0, _NCHUNK, chunk_body, 0)

    return _delinearize(_k(x_flat, tables_flat))
